# baseline (speedup 1.0000x reference)
import functools
import jax
import jax.numpy as jnp
from jax import lax
from jax.experimental import pallas as pl
from jax.experimental.pallas import tpu as pltpu
from jax.experimental.pallas import tpu_sc as plsc

N = 10000
NP = 10240
NPA = NP + 256
E = 160000
EPAD = 163840
G = 64
F = 256
NC = 2
NS = 16
NW = NC * NS
OWN = NPA // NW
ECH = 128
EPT = EPAD // NW
MAXCH = EPT // ECH + 2
CAP = MAXCH * ECH
DEFER = 16
NBLK = NP // 256
BSTR = 272


def _sc_binner_body(src_hbm, dst_hbm, lsrc_hbm, lloc_hbm, counts_hbm, sbuf,
                    dbuf, bins_s, bins_l, fb_s, fb_l, cnt_v, sem):
    c = lax.axis_index("c")
    s = lax.axis_index("s")
    p = s * NC + c
    pad_s = jnp.full((16,), N, jnp.int32)
    pad_l = jnp.zeros((16,), jnp.int32)
    iota16 = lax.iota(jnp.int32, 16)

    def ibin(i, _):
        bins_s[pl.ds(i * 16, 16)] = pad_s
        bins_l[pl.ds(i * 16, 16)] = pad_l
        return 0
    lax.fori_loop(0, NW * BSTR // 16, ibin, 0)

    def scan(blk, t):
        base = p * EPT + blk * 512
        pltpu.sync_copy(src_hbm.at[pl.ds(base, 512)], sbuf.at[pl.ds(0, 512)])
        pltpu.sync_copy(dst_hbm.at[pl.ds(base, 512)], dbuf.at[pl.ds(0, 512)])

        def edge(e, t):
            sv = sbuf[pl.ds(e, 16)][0]
            dv = dbuf[pl.ds(e, 16)][0]
            o = dv // OWN
            loc = dv - o * OWN
            k = t[0]
            for j in range(1, NW):
                k = jnp.where(o == j, t[j], k)
            slot = k & 255
            bins_s[pl.ds(o * BSTR + slot, 16)] = jnp.where(iota16 == 0, sv, N)
            bins_l[pl.ds(o * BSTR + slot, 16)] = jnp.where(iota16 == 0, loc, 0)
            kn = k + 1

            @pl.when((kn >= 128 + DEFER) & ((kn & 127) == DEFER))
            def _():
                m = (kn >> 7) - 1
                hbase = o * BSTR + (1 - ((kn >> 7) & 1)) * 128

                def stage(g, _):
                    fb_s[pl.ds(g * 16, 16)] = bins_s[pl.ds(hbase + g * 16, 16)]
                    fb_l[pl.ds(g * 16, 16)] = bins_l[pl.ds(hbase + g * 16, 16)]
                    return 0
                lax.fori_loop(0, 8, stage, 0)
                pltpu.sync_copy(fb_s, lsrc_hbm.at[p, o, pl.ds(m * ECH, ECH)])
                pltpu.sync_copy(fb_l, lloc_hbm.at[p, o, pl.ds(m * ECH, ECH)])

                def refill(g, _):
                    bins_s[pl.ds(hbase + g * 16, 16)] = pad_s
                    bins_l[pl.ds(hbase + g * 16, 16)] = pad_l
                    return 0
                lax.fori_loop(0, 8, refill, 0)

            return tuple(jnp.where(o == j, kn, t[j]) for j in range(NW))
        return lax.fori_loop(0, 512, edge, t)

    t = lax.fori_loop(0, EPT // 512, scan,
                      tuple(jnp.int32(0) for _ in range(NW)))

    for o in range(NW):
        to = t[o]
        fl = jnp.where(to >= 128 + DEFER, (to - DEFER) >> 7, 0)
        n = fl + jnp.where(to > fl * 128, 1, 0) \
               + jnp.where(to > fl * 128 + 128, 1, 0)
        cnt_v[pl.ds(o, 16)] = jnp.broadcast_to(n, (16,))

    pltpu.sync_copy(src_hbm.at[pl.ds(p * EPT, 512)], sbuf.at[pl.ds(0, 512)])

    for o in range(NW):
        to = t[o]
        fl = jnp.where(to >= 128 + DEFER, (to - DEFER) >> 7, 0)
        for q in range(2):
            mq = fl + q

            @pl.when(to > mq * 128)
            def _(o=o, mq=mq):
                hbase = o * BSTR + (mq & 1) * 128

                def stage(g, _):
                    fb_s[pl.ds(g * 16, 16)] = bins_s[pl.ds(hbase + g * 16, 16)]
                    fb_l[pl.ds(g * 16, 16)] = bins_l[pl.ds(hbase + g * 16, 16)]
                    return 0
                lax.fori_loop(0, 8, stage, 0)
                pltpu.sync_copy(fb_s, lsrc_hbm.at[p, o, pl.ds(mq * ECH, ECH)])
                pltpu.sync_copy(fb_l, lloc_hbm.at[p, o, pl.ds(mq * ECH, ECH)])

    pltpu.sync_copy(cnt_v.at[pl.ds(0, NW)], counts_hbm.at[pl.ds(p * NW, NW)])


def _sc_spmm_body(lsrc_hbm, lloc_hbm, counts_hbm, hs_hbm, acc_hbm, src_i,
                  loc_v, rows_v, acc_v, cnt_v, sem):
    c = lax.axis_index("c")
    s = lax.axis_index("s")
    o = s * NC + c

    z16 = jnp.zeros((16,), jnp.float32)

    def zr(i, _):
        acc_v[i // 16, pl.ds((i % 16) * 16, 16)] = z16
        return 0
    lax.fori_loop(0, OWN * 16, zr, 0)

    pltpu.sync_copy(counts_hbm, cnt_v.at[pl.ds(0, NW * NW)])

    def producer(p, _):
        nch = cnt_v[pl.ds(p * NW + o, 16)][0]

        def chunk(m, _):
            pltpu.sync_copy(lsrc_hbm.at[p, o, pl.ds(m * ECH, ECH)], src_i)
            pltpu.sync_copy(lloc_hbm.at[p, o, pl.ds(m * ECH, ECH)],
                            loc_v.at[pl.ds(0, ECH)])
            pltpu.async_copy(hs_hbm.at[src_i], rows_v, sem).wait()

            def edge(e, _):
                loc = loc_v[pl.ds(e, 16)][0]

                def cg(g, _):
                    x = rows_v[e, pl.ds(g * 16, 16)]
                    plsc.addupdate(acc_v.at[loc, pl.ds(g * 16, 16)], x)
                    return 0
                lax.fori_loop(0, F // 16, cg, 0)
                return 0
            lax.fori_loop(0, ECH, edge, 0)
            return 0
        lax.fori_loop(0, nch, chunk, 0)
        return 0
    lax.fori_loop(0, NW, producer, 0)

    pltpu.sync_copy(counts_hbm.at[pl.ds(0, NW)], cnt_v.at[pl.ds(0, NW)])
    pltpu.sync_copy(acc_v, acc_hbm.at[pl.ds(o * OWN, OWN)])


@functools.lru_cache(maxsize=1)
def _sc_kernels():
    mesh = plsc.VectorSubcoreMesh(
        core_axis_name="c", subcore_axis_name="s",
        num_cores=NC, num_subcores=NS)
    binner = pl.kernel(
        _sc_binner_body,
        out_type=[jax.ShapeDtypeStruct((NW, NW, CAP), jnp.int32),
                  jax.ShapeDtypeStruct((NW, NW, CAP), jnp.int32),
                  jax.ShapeDtypeStruct((NW * NW,), jnp.int32)],
        mesh=mesh,
        scratch_types=[
            pltpu.VMEM((512 + 16,), jnp.int32),
            pltpu.VMEM((512 + 16,), jnp.int32),
            pltpu.VMEM((NW * BSTR,), jnp.int32),
            pltpu.VMEM((NW * BSTR,), jnp.int32),
            pltpu.VMEM((ECH,), jnp.int32),
            pltpu.VMEM((ECH,), jnp.int32),
            pltpu.VMEM((NW + 16,), jnp.int32),
            pltpu.SemaphoreType.DMA,
        ],
    )
    spmm = pl.kernel(
        _sc_spmm_body,
        out_type=jax.ShapeDtypeStruct((NPA, F), jnp.float32),
        mesh=mesh,
        scratch_types=[
            pltpu.VMEM((ECH,), jnp.int32),
            pltpu.VMEM((ECH + 16,), jnp.int32),
            pltpu.VMEM((ECH, F), jnp.float32),
            pltpu.VMEM((OWN, F), jnp.float32),
            pltpu.VMEM((NW * NW + 16,), jnp.int32),
            pltpu.SemaphoreType.DMA,
        ],
    )
    return binner, spmm


def _sc_bin(src, dst):
    return _sc_kernels()[0](src, dst)


def _sc_spmm(lsrc, lloc, counts, hs):
    return _sc_kernels()[1](lsrc, lloc, counts, hs)


def _ident():
    r = lax.broadcasted_iota(jnp.int32, (256, 256), 0)
    l = lax.broadcasted_iota(jnp.int32, (256, 256), 1)
    return (r == l).astype(jnp.float32)


def _diag(dis_row):
    return _ident() * jnp.broadcast_to(dis_row, (256, 256))


def _k1_body(dg_ref, x_ref, w_ref, hs_ref, dis_ref):
    ones_row = jnp.ones((1, 256), jnp.float32)
    deg = jnp.dot(ones_row, dg_ref[...] * _ident(),
                  preferred_element_type=jnp.float32)
    col = lax.broadcasted_iota(jnp.int32, (1, 256), 1) + pl.program_id(0) * 256
    dis_row = jnp.where(col < N, lax.rsqrt(deg + 1.0), 0.0)
    dis_ref[...] = dis_row.reshape(1, 1, 256)
    d = _diag(dis_row)
    xw = jnp.dot(x_ref[...], w_ref[...], preferred_element_type=jnp.float32)
    hs_ref[...] = jnp.dot(d, xw, preferred_element_type=jnp.float32)


def _k2_body(dis_ref, acc_ref, hs_ref, b_ref, w_ref, out_ref):
    dis_row = dis_ref[...].reshape(1, 256)
    d = _diag(dis_row)
    pre = jnp.dot(d, acc_ref[...] + hs_ref[...],
                  preferred_element_type=jnp.float32) + b_ref[...]
    x = jnp.maximum(pre, 0.0)
    xw = jnp.dot(x, w_ref[...], preferred_element_type=jnp.float32)
    out_ref[...] = jnp.dot(d, xw, preferred_element_type=jnp.float32)


def _k3_body(dis_ref, acc_ref, hs_ref, b_ref, bat_ref, pool_ref):
    dis_row = dis_ref[...].reshape(1, 256)
    d = _diag(dis_row)
    pre = jnp.dot(d, acc_ref[...] + hs_ref[...],
                  preferred_element_type=jnp.float32) + b_ref[...]
    x = jnp.maximum(pre, 0.0)
    bat = bat_ref[...].reshape(1, 256)
    gi = lax.broadcasted_iota(jnp.int32, (G, 256), 0)
    oh = (gi == jnp.broadcast_to(bat, (G, 256))).astype(jnp.float32)
    blk = jnp.dot(oh, x, preferred_element_type=jnp.float32)

    @pl.when(pl.program_id(0) == 0)
    def _():
        pool_ref[...] = jnp.zeros_like(pool_ref)
    pool_ref[...] += blk


def _k4_body(pool_ref, w1_ref, b1_ref, w2_ref, b2_ref, out_ref):
    g1 = jnp.maximum(
        jnp.dot(pool_ref[...], w1_ref[...],
                preferred_element_type=jnp.float32) + b1_ref[...], 0.0)
    out_ref[...] = jnp.dot(g1, w2_ref[...],
                           preferred_element_type=jnp.float32) + b2_ref[...]


_rowspec = pl.BlockSpec((256, F), lambda i: (i, 0))
_disspec = pl.BlockSpec((1, 1, 256), lambda i: (i, 0, 0))
_constw = pl.BlockSpec((F, F), lambda i: (0, 0))
_constb = pl.BlockSpec((1, F), lambda i: (0, 0))

_k1 = pl.pallas_call(
    _k1_body,
    grid=(NBLK,),
    in_specs=[_rowspec, _rowspec, _constw],
    out_specs=[_rowspec, _disspec],
    out_shape=[jax.ShapeDtypeStruct((NP, F), jnp.float32),
               jax.ShapeDtypeStruct((NBLK, 1, 256), jnp.float32)],
)

_k2 = pl.pallas_call(
    _k2_body,
    grid=(NBLK,),
    in_specs=[_disspec, _rowspec, _rowspec, _constb, _constw],
    out_specs=_rowspec,
    out_shape=jax.ShapeDtypeStruct((NP, F), jnp.float32),
)

_k3 = pl.pallas_call(
    _k3_body,
    grid=(NBLK,),
    in_specs=[_disspec, _rowspec, _rowspec, _constb, _disspec],
    out_specs=pl.BlockSpec((G, F), lambda i: (0, 0)),
    out_shape=jax.ShapeDtypeStruct((G, F), jnp.float32),
)

_k4 = pl.pallas_call(
    _k4_body,
    in_specs=[pl.BlockSpec((G, F), lambda: (0, 0)),
              pl.BlockSpec((F, F), lambda: (0, 0)),
              pl.BlockSpec((1, F), lambda: (0, 0)),
              pl.BlockSpec((F, 128), lambda: (0, 0)),
              pl.BlockSpec((1, 128), lambda: (0, 0))],
    out_specs=pl.BlockSpec((G, 128), lambda: (0, 0)),
    out_shape=jax.ShapeDtypeStruct((G, 128), jnp.float32),
)


def kernel(x, edge_index, batch, W1, b1, W2, b2, W3, b3, W4, b4,
           fc1_w, fc1_b, out_w, out_b):
    f32 = jnp.float32
    xp = jnp.zeros((NP, F), f32).at[:N].set(x)
    src = jnp.concatenate(
        [edge_index[0], jnp.full((EPAD - E,), N, jnp.int32)]).astype(jnp.int32)
    dst = jnp.concatenate(
        [edge_index[1], jnp.full((EPAD - E,), NP, jnp.int32)]).astype(jnp.int32)
    bat3 = jnp.concatenate(
        [batch.astype(jnp.int32), jnp.full((NP - N,), G, jnp.int32)]
    ).reshape(NBLK, 1, 256)
    row_id = lax.broadcasted_iota(jnp.int32, (NP, 1), 0)
    ones_p = jnp.where(row_id < N, 1.0, 0.0) * jnp.ones((NP, F), f32)

    lsrc, lloc, counts = _sc_bin(src, dst)

    deg_acc = _sc_spmm(lsrc, lloc, counts, ones_p)
    hs, dis3 = _k1(deg_acc, xp, W1)
    for (W, b) in ((W2, b1), (W3, b2), (W4, b3)):
        acc = _sc_spmm(lsrc, lloc, counts, hs)
        hs = _k2(dis3, acc, hs, b.reshape(1, F), W)
    acc = _sc_spmm(lsrc, lloc, counts, hs)
    pooled = _k3(dis3, acc, hs, b4.reshape(1, F), bat3)

    ow = jnp.zeros((F, 128), f32).at[:, 0].set(out_w[:, 0])
    ob = jnp.zeros((1, 128), f32).at[0, 0].set(out_b[0])
    out = _k4(pooled, fc1_w, fc1_b.reshape(1, F), ow, ob)
    return out[:, :1]


# speedup vs baseline: 1.0063x; 1.0063x over previous
import functools
import jax
import jax.numpy as jnp
from jax import lax
from jax.experimental import pallas as pl
from jax.experimental.pallas import tpu as pltpu
from jax.experimental.pallas import tpu_sc as plsc

N = 10000
NP = 10240
NPA = NP + 256
E = 160000
EPAD = 163840
G = 64
F = 256
NC = 2
NS = 16
NW = NC * NS
OWN = NPA // NW
ECH = 128
EPT = EPAD // NW
MAXCH = EPT // ECH + 2
CAP = MAXCH * ECH
DEFER = 16
NBLK = NP // 256
BSTR = 272


def _sc_binner_body(src_hbm, dst_hbm, lsrc_hbm, lloc_hbm, counts_hbm, sbuf,
                    dbuf, bins_s, bins_l, fb_s, fb_l, cnt_v, sem):
    c = lax.axis_index("c")
    s = lax.axis_index("s")
    p = s * NC + c
    pad_s = jnp.full((16,), N, jnp.int32)
    pad_l = jnp.zeros((16,), jnp.int32)
    iota16 = lax.iota(jnp.int32, 16)

    def ibin(i, _):
        bins_s[pl.ds(i * 16, 16)] = pad_s
        bins_l[pl.ds(i * 16, 16)] = pad_l
        return 0
    lax.fori_loop(0, NW * BSTR // 16, ibin, 0)

    def scan(blk, t):
        base = p * EPT + blk * 512
        pltpu.sync_copy(src_hbm.at[pl.ds(base, 512)], sbuf.at[pl.ds(0, 512)])
        pltpu.sync_copy(dst_hbm.at[pl.ds(base, 512)], dbuf.at[pl.ds(0, 512)])

        def edge(e, t):
            sv = sbuf[pl.ds(e, 16)][0]
            dv = dbuf[pl.ds(e, 16)][0]
            o = dv // OWN
            loc = dv - o * OWN
            k = t[0]
            for j in range(1, NW):
                k = jnp.where(o == j, t[j], k)
            slot = k & 255
            bins_s[pl.ds(o * BSTR + slot, 16)] = jnp.where(iota16 == 0, sv, N)
            bins_l[pl.ds(o * BSTR + slot, 16)] = jnp.where(iota16 == 0, loc, 0)
            kn = k + 1

            @pl.when((kn >= 128 + DEFER) & ((kn & 127) == DEFER))
            def _():
                m = (kn >> 7) - 1
                hbase = o * BSTR + (1 - ((kn >> 7) & 1)) * 128

                for g in range(8):
                    fb_s[pl.ds(g * 16, 16)] = bins_s[pl.ds(hbase + g * 16, 16)]
                    fb_l[pl.ds(g * 16, 16)] = bins_l[pl.ds(hbase + g * 16, 16)]
                pltpu.sync_copy(fb_s, lsrc_hbm.at[p, o, pl.ds(m * ECH, ECH)])
                pltpu.sync_copy(fb_l, lloc_hbm.at[p, o, pl.ds(m * ECH, ECH)])
                for g in range(8):
                    bins_s[pl.ds(hbase + g * 16, 16)] = pad_s
                    bins_l[pl.ds(hbase + g * 16, 16)] = pad_l

            return tuple(jnp.where(o == j, kn, t[j]) for j in range(NW))
        return lax.fori_loop(0, 512, edge, t)

    t = lax.fori_loop(0, EPT // 512, scan,
                      tuple(jnp.int32(0) for _ in range(NW)))

    for o in range(NW):
        to = t[o]
        fl = jnp.where(to >= 128 + DEFER, (to - DEFER) >> 7, 0)
        n = fl + jnp.where(to > fl * 128, 1, 0) \
               + jnp.where(to > fl * 128 + 128, 1, 0)
        cnt_v[pl.ds(o, 16)] = jnp.broadcast_to(n, (16,))

    pltpu.sync_copy(src_hbm.at[pl.ds(p * EPT, 512)], sbuf.at[pl.ds(0, 512)])

    for o in range(NW):
        to = t[o]
        fl = jnp.where(to >= 128 + DEFER, (to - DEFER) >> 7, 0)
        for q in range(2):
            mq = fl + q

            @pl.when(to > mq * 128)
            def _(o=o, mq=mq):
                hbase = o * BSTR + (mq & 1) * 128

                for g in range(8):
                    fb_s[pl.ds(g * 16, 16)] = bins_s[pl.ds(hbase + g * 16, 16)]
                    fb_l[pl.ds(g * 16, 16)] = bins_l[pl.ds(hbase + g * 16, 16)]
                pltpu.sync_copy(fb_s, lsrc_hbm.at[p, o, pl.ds(mq * ECH, ECH)])
                pltpu.sync_copy(fb_l, lloc_hbm.at[p, o, pl.ds(mq * ECH, ECH)])

    pltpu.sync_copy(cnt_v.at[pl.ds(0, NW)], counts_hbm.at[pl.ds(p * NW, NW)])


def _sc_spmm_body(lsrc_hbm, lloc_hbm, counts_hbm, hs_hbm, acc_hbm, src_i,
                  loc_v, rows_v, acc_v, cnt_v, sem):
    c = lax.axis_index("c")
    s = lax.axis_index("s")
    o = s * NC + c

    z16 = jnp.zeros((16,), jnp.float32)

    def zr(i, _):
        for g in range(F // 16):
            acc_v[i, pl.ds(g * 16, 16)] = z16
        return 0
    lax.fori_loop(0, OWN, zr, 0)

    pltpu.sync_copy(counts_hbm, cnt_v.at[pl.ds(0, NW * NW)])

    def producer(p, _):
        nch = cnt_v[pl.ds(p * NW + o, 16)][0]

        def chunk(m, _):
            pltpu.sync_copy(lsrc_hbm.at[p, o, pl.ds(m * ECH, ECH)], src_i)
            pltpu.sync_copy(lloc_hbm.at[p, o, pl.ds(m * ECH, ECH)],
                            loc_v.at[pl.ds(0, ECH)])
            pltpu.async_copy(hs_hbm.at[src_i], rows_v, sem).wait()

            def edge(e, _):
                loc = loc_v[pl.ds(e, 16)][0]
                for g in range(F // 16):
                    x = rows_v[e, pl.ds(g * 16, 16)]
                    plsc.addupdate(acc_v.at[loc, pl.ds(g * 16, 16)], x)
                return 0
            lax.fori_loop(0, ECH, edge, 0, unroll=2)
            return 0
        lax.fori_loop(0, nch, chunk, 0)
        return 0
    lax.fori_loop(0, NW, producer, 0)

    pltpu.sync_copy(counts_hbm.at[pl.ds(0, NW)], cnt_v.at[pl.ds(0, NW)])
    pltpu.sync_copy(acc_v, acc_hbm.at[pl.ds(o * OWN, OWN)])


@functools.lru_cache(maxsize=1)
def _sc_kernels():
    mesh = plsc.VectorSubcoreMesh(
        core_axis_name="c", subcore_axis_name="s",
        num_cores=NC, num_subcores=NS)
    binner = pl.kernel(
        _sc_binner_body,
        out_type=[jax.ShapeDtypeStruct((NW, NW, CAP), jnp.int32),
                  jax.ShapeDtypeStruct((NW, NW, CAP), jnp.int32),
                  jax.ShapeDtypeStruct((NW * NW,), jnp.int32)],
        mesh=mesh,
        scratch_types=[
            pltpu.VMEM((512 + 16,), jnp.int32),
            pltpu.VMEM((512 + 16,), jnp.int32),
            pltpu.VMEM((NW * BSTR,), jnp.int32),
            pltpu.VMEM((NW * BSTR,), jnp.int32),
            pltpu.VMEM((ECH,), jnp.int32),
            pltpu.VMEM((ECH,), jnp.int32),
            pltpu.VMEM((NW + 16,), jnp.int32),
            pltpu.SemaphoreType.DMA,
        ],
    )
    spmm = pl.kernel(
        _sc_spmm_body,
        out_type=jax.ShapeDtypeStruct((NPA, F), jnp.float32),
        mesh=mesh,
        scratch_types=[
            pltpu.VMEM((ECH,), jnp.int32),
            pltpu.VMEM((ECH + 16,), jnp.int32),
            pltpu.VMEM((ECH, F), jnp.float32),
            pltpu.VMEM((OWN, F), jnp.float32),
            pltpu.VMEM((NW * NW + 16,), jnp.int32),
            pltpu.SemaphoreType.DMA,
        ],
    )
    return binner, spmm


def _sc_bin(src, dst):
    return _sc_kernels()[0](src, dst)


def _sc_spmm(lsrc, lloc, counts, hs):
    return _sc_kernels()[1](lsrc, lloc, counts, hs)


def _ident():
    r = lax.broadcasted_iota(jnp.int32, (256, 256), 0)
    l = lax.broadcasted_iota(jnp.int32, (256, 256), 1)
    return (r == l).astype(jnp.float32)


def _diag(dis_row):
    return _ident() * jnp.broadcast_to(dis_row, (256, 256))


def _k1_body(dg_ref, x_ref, w_ref, hs_ref, dis_ref):
    ones_row = jnp.ones((1, 256), jnp.float32)
    deg = jnp.dot(ones_row, dg_ref[...] * _ident(),
                  preferred_element_type=jnp.float32)
    col = lax.broadcasted_iota(jnp.int32, (1, 256), 1) + pl.program_id(0) * 256
    dis_row = jnp.where(col < N, lax.rsqrt(deg + 1.0), 0.0)
    dis_ref[...] = dis_row.reshape(1, 1, 256)
    d = _diag(dis_row)
    xw = jnp.dot(x_ref[...], w_ref[...], preferred_element_type=jnp.float32)
    hs_ref[...] = jnp.dot(d, xw, preferred_element_type=jnp.float32)


def _k2_body(dis_ref, acc_ref, hs_ref, b_ref, w_ref, out_ref):
    dis_row = dis_ref[...].reshape(1, 256)
    d = _diag(dis_row)
    pre = jnp.dot(d, acc_ref[...] + hs_ref[...],
                  preferred_element_type=jnp.float32) + b_ref[...]
    x = jnp.maximum(pre, 0.0)
    xw = jnp.dot(x, w_ref[...], preferred_element_type=jnp.float32)
    out_ref[...] = jnp.dot(d, xw, preferred_element_type=jnp.float32)


def _k3_body(dis_ref, acc_ref, hs_ref, b_ref, bat_ref, pool_ref):
    dis_row = dis_ref[...].reshape(1, 256)
    d = _diag(dis_row)
    pre = jnp.dot(d, acc_ref[...] + hs_ref[...],
                  preferred_element_type=jnp.float32) + b_ref[...]
    x = jnp.maximum(pre, 0.0)
    bat = bat_ref[...].reshape(1, 256)
    gi = lax.broadcasted_iota(jnp.int32, (G, 256), 0)
    oh = (gi == jnp.broadcast_to(bat, (G, 256))).astype(jnp.float32)
    blk = jnp.dot(oh, x, preferred_element_type=jnp.float32)

    @pl.when(pl.program_id(0) == 0)
    def _():
        pool_ref[...] = jnp.zeros_like(pool_ref)
    pool_ref[...] += blk


def _k4_body(pool_ref, w1_ref, b1_ref, w2_ref, b2_ref, out_ref):
    g1 = jnp.maximum(
        jnp.dot(pool_ref[...], w1_ref[...],
                preferred_element_type=jnp.float32) + b1_ref[...], 0.0)
    out_ref[...] = jnp.dot(g1, w2_ref[...],
                           preferred_element_type=jnp.float32) + b2_ref[...]


_rowspec = pl.BlockSpec((256, F), lambda i: (i, 0))
_disspec = pl.BlockSpec((1, 1, 256), lambda i: (i, 0, 0))
_constw = pl.BlockSpec((F, F), lambda i: (0, 0))
_constb = pl.BlockSpec((1, F), lambda i: (0, 0))

_k1 = pl.pallas_call(
    _k1_body,
    grid=(NBLK,),
    in_specs=[_rowspec, _rowspec, _constw],
    out_specs=[_rowspec, _disspec],
    out_shape=[jax.ShapeDtypeStruct((NP, F), jnp.float32),
               jax.ShapeDtypeStruct((NBLK, 1, 256), jnp.float32)],
)

_k2 = pl.pallas_call(
    _k2_body,
    grid=(NBLK,),
    in_specs=[_disspec, _rowspec, _rowspec, _constb, _constw],
    out_specs=_rowspec,
    out_shape=jax.ShapeDtypeStruct((NP, F), jnp.float32),
)

_k3 = pl.pallas_call(
    _k3_body,
    grid=(NBLK,),
    in_specs=[_disspec, _rowspec, _rowspec, _constb, _disspec],
    out_specs=pl.BlockSpec((G, F), lambda i: (0, 0)),
    out_shape=jax.ShapeDtypeStruct((G, F), jnp.float32),
)

_k4 = pl.pallas_call(
    _k4_body,
    in_specs=[pl.BlockSpec((G, F), lambda: (0, 0)),
              pl.BlockSpec((F, F), lambda: (0, 0)),
              pl.BlockSpec((1, F), lambda: (0, 0)),
              pl.BlockSpec((F, 128), lambda: (0, 0)),
              pl.BlockSpec((1, 128), lambda: (0, 0))],
    out_specs=pl.BlockSpec((G, 128), lambda: (0, 0)),
    out_shape=jax.ShapeDtypeStruct((G, 128), jnp.float32),
)


def kernel(x, edge_index, batch, W1, b1, W2, b2, W3, b3, W4, b4,
           fc1_w, fc1_b, out_w, out_b):
    f32 = jnp.float32
    xp = jnp.zeros((NP, F), f32).at[:N].set(x)
    src = jnp.concatenate(
        [edge_index[0], jnp.full((EPAD - E,), N, jnp.int32)]).astype(jnp.int32)
    dst = jnp.concatenate(
        [edge_index[1], jnp.full((EPAD - E,), NP, jnp.int32)]).astype(jnp.int32)
    bat3 = jnp.concatenate(
        [batch.astype(jnp.int32), jnp.full((NP - N,), G, jnp.int32)]
    ).reshape(NBLK, 1, 256)
    row_id = lax.broadcasted_iota(jnp.int32, (NP, 1), 0)
    ones_p = jnp.where(row_id < N, 1.0, 0.0) * jnp.ones((NP, F), f32)

    lsrc, lloc, counts = _sc_bin(src, dst)

    deg_acc = _sc_spmm(lsrc, lloc, counts, ones_p)
    hs, dis3 = _k1(deg_acc, xp, W1)
    for (W, b) in ((W2, b1), (W3, b2), (W4, b3)):
        acc = _sc_spmm(lsrc, lloc, counts, hs)
        hs = _k2(dis3, acc, hs, b.reshape(1, F), W)
    acc = _sc_spmm(lsrc, lloc, counts, hs)
    pooled = _k3(dis3, acc, hs, b4.reshape(1, F), bat3)

    ow = jnp.zeros((F, 128), f32).at[:, 0].set(out_w[:, 0])
    ob = jnp.zeros((1, 128), f32).at[0, 0].set(out_b[0])
    out = _k4(pooled, fc1_w, fc1_b.reshape(1, F), ow, ob)
    return out[:, :1]
